# Initial kernel scaffold; baseline (speedup 1.0000x reference)
import jax
import jax.numpy as jnp
from jax.experimental import pallas as pl


def kernel(x, edge_index, norm, W1, b1, W2, b2):
    raise NotImplementedError("write your pallas kernel here")


# speedup vs baseline: 9.2682x; 9.2682x over previous
import functools

import jax
import jax.numpy as jnp
from jax import lax
from jax.experimental import pallas as pl
from jax.experimental.pallas import tpu as pltpu
from jax.experimental.pallas import tpu_sc as plsc

N, E, F, H, C, K = 10000, 160000, 256, 512, 64, 8

NUM_TILES = 16
CHUNK = 80
NCH = E // NUM_TILES // CHUNK
NPAD = 10240
ROWS_PER_TILE = NPAD // NUM_TILES
CH = C // 2
VEC = 16


def _mlp(x, W1t, b1, W2t, b2):
    Bm = 2000

    def body(x_ref, w1_ref, b1_ref, w2_ref, b2_ref, o_ref):
        h = jnp.dot(x_ref[...], w1_ref[...],
                    preferred_element_type=jnp.float32,
                    precision=lax.Precision.HIGHEST)
        h = jnp.maximum(h + b1_ref[...], 0.0)
        o_ref[...] = jnp.dot(h, w2_ref[...],
                             preferred_element_type=jnp.float32,
                             precision=lax.Precision.HIGHEST) + b2_ref[...]

    return pl.pallas_call(
        body,
        grid=(N // Bm,),
        in_specs=[
            pl.BlockSpec((Bm, F), lambda i: (i, 0)),
            pl.BlockSpec((F, H), lambda i: (0, 0)),
            pl.BlockSpec((1, H), lambda i: (0, 0)),
            pl.BlockSpec((H, C), lambda i: (0, 0)),
            pl.BlockSpec((1, C), lambda i: (0, 0)),
        ],
        out_specs=pl.BlockSpec((Bm, C), lambda i: (i, 0)),
        out_shape=jax.ShapeDtypeStruct((N, C), jnp.float32),
    )(x, W1t, b1.reshape(1, H), W2t, b2.reshape(1, C))


def _log_softmax(h):
    Bm = 2000

    def body(h_ref, o_ref):
        v = h_ref[...]
        m = jnp.max(v, axis=1, keepdims=True)
        e = jnp.exp(v - m)
        s = jnp.log(jnp.sum(e, axis=1, keepdims=True))
        o_ref[...] = v - m - s

    return pl.pallas_call(
        body,
        grid=(N // Bm,),
        in_specs=[pl.BlockSpec((Bm, C), lambda i: (i, 0))],
        out_specs=pl.BlockSpec((Bm, C), lambda i: (i, 0)),
        out_shape=jax.ShapeDtypeStruct((N, C), jnp.float32),
    )(h)


def _make_prop_kernel():
    mesh = plsc.VectorSubcoreMesh(core_axis_name="c", subcore_axis_name="s")

    @functools.partial(
        pl.kernel,
        mesh=mesh,
        compiler_params=pltpu.CompilerParams(use_tc_tiling_on_sc=False),
        out_type=jax.ShapeDtypeStruct((2, NPAD, CH), jnp.float32),
        scratch_types=[
            pltpu.VMEM_SHARED((NPAD, CH), jnp.float32),
            pltpu.VMEM_SHARED((NPAD, CH), jnp.float32),
            pltpu.VMEM((NCH, CHUNK), jnp.int32),
            pltpu.VMEM((NCH, CHUNK), jnp.int32),
            pltpu.VMEM((NCH, CHUNK), jnp.float32),
            pltpu.VMEM((CHUNK, CH), jnp.float32),
            pltpu.VMEM((ROWS_PER_TILE, CH), jnp.float32),
        ],
    )
    def prop(h0_hbm, src_hbm, dst_hbm, norm_hbm, out_hbm,
             bufA, bufB, src_v, dst_v, norm_v, rows, stage):
        cid = lax.axis_index("c")
        sid = lax.axis_index("s")
        row0 = sid * ROWS_PER_TILE
        my_rows = pl.ds(row0, ROWS_PER_TILE)

        pltpu.sync_copy(src_hbm.at[sid], src_v)
        pltpu.sync_copy(dst_hbm.at[sid], dst_v)
        pltpu.sync_copy(norm_hbm.at[sid], norm_v)

        pltpu.sync_copy(h0_hbm.at[cid, my_rows], stage)
        pltpu.sync_copy(stage, bufA.at[my_rows])

        zvec = jnp.zeros((VEC,), jnp.float32)

        def zrow(i, _):
            for j in range(CH // VEC):
                stage[i, pl.ds(j * VEC, VEC)] = zvec
            return 0

        lax.fori_loop(0, ROWS_PER_TILE, zrow, 0)
        pltpu.sync_copy(stage, bufB.at[my_rows])
        plsc.subcore_barrier()

        vecs_per_row = CH // VEC

        def one_round(rd, wr):
            def chunk_body(ci, _):
                pltpu.sync_copy(rd.at[src_v.at[ci]], rows)

                for g in range(CHUNK // VEC):
                    nvec = norm_v[ci, pl.ds(g * VEC, VEC)]
                    for j in range(VEC * vecs_per_row):
                        r = g * VEC + j // vecs_per_row
                        c0 = (j % vecs_per_row) * VEC
                        nv = nvec[j // vecs_per_row]
                        sl = pl.ds(c0, VEC)
                        rows[r, sl] = rows[r, sl] * nv

                pltpu.sync_copy(rows, wr.at[dst_v.at[ci]], add=True)
                return 0

            lax.fori_loop(0, NCH, chunk_body, 0)
            plsc.subcore_barrier()

        def round_pair(rp, _):
            one_round(bufA, bufB)
            pltpu.sync_copy(stage, bufA.at[my_rows])
            plsc.subcore_barrier()
            one_round(bufB, bufA)
            pltpu.sync_copy(stage, bufB.at[my_rows])
            plsc.subcore_barrier()
            return 0

        lax.fori_loop(0, K // 2, round_pair, 0)

        pltpu.sync_copy(bufA.at[my_rows], stage)
        pltpu.sync_copy(stage, out_hbm.at[cid, my_rows])

    return prop


_prop = _make_prop_kernel()


def kernel(x, edge_index, norm, W1, b1, W2, b2):
    h0 = _mlp(x, W1.T, b1, W2.T, b2)

    h0s = jnp.stack([h0[:, :CH], h0[:, CH:]], axis=0)
    h0s = jnp.pad(h0s, ((0, 0), (0, NPAD - N), (0, 0)))
    srcr = edge_index[0].reshape(NUM_TILES, NCH, CHUNK)
    dstr = edge_index[1].reshape(NUM_TILES, NCH, CHUNK)
    normr = norm.reshape(NUM_TILES, NCH, CHUNK)

    hks = _prop(h0s, srcr, dstr, normr)
    hk = jnp.concatenate([hks[0, :N], hks[1, :N]], axis=1)
    return _log_softmax(hk)


# speedup vs baseline: 13.3957x; 1.4453x over previous
import functools

import jax
import jax.numpy as jnp
from jax import lax
from jax.experimental import pallas as pl
from jax.experimental.pallas import tpu as pltpu
from jax.experimental.pallas import tpu_sc as plsc

N, E, F, H, C, K = 10000, 160000, 256, 512, 64, 8

NUM_TILES = 16
CHUNK = 128
NCH = 81
EPT = NCH * CHUNK
EPAD = NUM_TILES * EPT
NPAD = 10240
ROWS_PER_TILE = NPAD // NUM_TILES
CH = C // 2
VEC = 16


def _mlp(x, W1t, b1, W2t, b2):
    Bm = 2000

    def body(x_ref, w1_ref, b1_ref, w2_ref, b2_ref, o_ref):
        h = jnp.dot(x_ref[...], w1_ref[...],
                    preferred_element_type=jnp.float32,
                    precision=lax.Precision.HIGHEST)
        h = jnp.maximum(h + b1_ref[...], 0.0)
        o_ref[...] = jnp.dot(h, w2_ref[...],
                             preferred_element_type=jnp.float32,
                             precision=lax.Precision.HIGHEST) + b2_ref[...]

    return pl.pallas_call(
        body,
        grid=(N // Bm,),
        in_specs=[
            pl.BlockSpec((Bm, F), lambda i: (i, 0)),
            pl.BlockSpec((F, H), lambda i: (0, 0)),
            pl.BlockSpec((1, H), lambda i: (0, 0)),
            pl.BlockSpec((H, C), lambda i: (0, 0)),
            pl.BlockSpec((1, C), lambda i: (0, 0)),
        ],
        out_specs=pl.BlockSpec((Bm, C), lambda i: (i, 0)),
        out_shape=jax.ShapeDtypeStruct((N, C), jnp.float32),
    )(x, W1t, b1.reshape(1, H), W2t, b2.reshape(1, C))


def _log_softmax(h):
    Bm = 2000

    def body(h_ref, o_ref):
        v = h_ref[...]
        m = jnp.max(v, axis=1, keepdims=True)
        e = jnp.exp(v - m)
        s = jnp.log(jnp.sum(e, axis=1, keepdims=True))
        o_ref[...] = v - m - s

    return pl.pallas_call(
        body,
        grid=(N // Bm,),
        in_specs=[pl.BlockSpec((Bm, C), lambda i: (i, 0))],
        out_specs=pl.BlockSpec((Bm, C), lambda i: (i, 0)),
        out_shape=jax.ShapeDtypeStruct((N, C), jnp.float32),
    )(h)


def _make_prop_kernel():
    mesh = plsc.VectorSubcoreMesh(core_axis_name="c", subcore_axis_name="s")

    @functools.partial(
        pl.kernel,
        mesh=mesh,
        compiler_params=pltpu.CompilerParams(use_tc_tiling_on_sc=False),
        out_type=jax.ShapeDtypeStruct((2, NPAD, CH), jnp.float32),
        scratch_types=[
            pltpu.VMEM_SHARED((NPAD, CH), jnp.float32),
            pltpu.VMEM_SHARED((NPAD, CH), jnp.float32),
            pltpu.VMEM((NCH, CHUNK), jnp.int32),
            pltpu.VMEM((NCH, CHUNK), jnp.int32),
            pltpu.VMEM((NCH, CHUNK), jnp.float32),
            pltpu.VMEM((CHUNK, CH), jnp.float32),
            pltpu.VMEM((CHUNK, CH), jnp.float32),
            pltpu.VMEM((CHUNK, CH), jnp.float32),
            pltpu.VMEM((ROWS_PER_TILE, CH), jnp.float32),
            pltpu.SemaphoreType.DMA,
            pltpu.SemaphoreType.DMA,
            pltpu.SemaphoreType.DMA,
            pltpu.SemaphoreType.DMA,
            pltpu.SemaphoreType.DMA,
            pltpu.SemaphoreType.DMA,
        ],
    )
    def prop(h0_hbm, src_hbm, dst_hbm, norm_hbm, out_hbm,
             bufA, bufB, src_v, dst_v, norm_v, rows0, rows1, rows2, stage,
             gsem0, gsem1, gsem2, ssem0, ssem1, ssem2):
        rows_bufs = (rows0, rows1, rows2)
        gsems = (gsem0, gsem1, gsem2)
        ssems = (ssem0, ssem1, ssem2)
        cid = lax.axis_index("c")
        sid = lax.axis_index("s")
        row0 = sid * ROWS_PER_TILE
        my_rows = pl.ds(row0, ROWS_PER_TILE)

        pltpu.sync_copy(src_hbm.at[sid], src_v)
        pltpu.sync_copy(dst_hbm.at[sid], dst_v)
        pltpu.sync_copy(norm_hbm.at[sid], norm_v)

        pltpu.sync_copy(h0_hbm.at[cid, my_rows], stage)
        pltpu.sync_copy(stage, bufA.at[my_rows])

        zvec = jnp.zeros((VEC,), jnp.float32)

        def zrow(i, _):
            for j in range(CH // VEC):
                stage[i, pl.ds(j * VEC, VEC)] = zvec
            return 0

        lax.fori_loop(0, ROWS_PER_TILE, zrow, 0)
        pltpu.sync_copy(stage, bufB.at[my_rows])
        plsc.subcore_barrier()

        vecs_per_row = CH // VEC

        def scale(rows, ci):
            for g in range(CHUNK // VEC):
                nvec = norm_v[ci, pl.ds(g * VEC, VEC)]
                for j in range(VEC * vecs_per_row):
                    r = g * VEC + j // vecs_per_row
                    c0 = (j % vecs_per_row) * VEC
                    nv = nvec[j // vecs_per_row]
                    sl = pl.ds(c0, VEC)
                    rows[r, sl] = rows[r, sl] * nv

        def one_round(rd, wr):
            pltpu.make_async_copy(rd.at[src_v.at[0]], rows0, gsem0).start()

            @pl.loop(0, NCH, step=3)
            def chunk_trio(c):
                for k in range(3):
                    cc = c + k
                    b = k
                    nb = (k + 1) % 3
                    @pl.when(cc >= 2)
                    def _():
                        pltpu.make_async_copy(
                            rows_bufs[nb], wr.at[dst_v.at[cc - 2]],
                            ssems[nb]).wait()
                    @pl.when(cc + 1 < NCH)
                    def _():
                        pltpu.make_async_copy(
                            rd.at[src_v.at[cc + 1]], rows_bufs[nb],
                            gsems[nb]).start()
                    pltpu.make_async_copy(
                        rd.at[src_v.at[cc]], rows_bufs[b], gsems[b]).wait()
                    scale(rows_bufs[b], cc)
                    pltpu.make_async_copy(
                        rows_bufs[b], wr.at[dst_v.at[cc]],
                        ssems[b]).start(add=True)

            pltpu.make_async_copy(
                rows_bufs[(NCH - 2) % 3], wr.at[dst_v.at[NCH - 2]],
                ssems[(NCH - 2) % 3]).wait()
            pltpu.make_async_copy(
                rows_bufs[(NCH - 1) % 3], wr.at[dst_v.at[NCH - 1]],
                ssems[(NCH - 1) % 3]).wait()
            plsc.subcore_barrier()

        def round_pair(rp, _):
            one_round(bufA, bufB)
            pltpu.sync_copy(stage, bufA.at[my_rows])
            plsc.subcore_barrier()
            one_round(bufB, bufA)
            pltpu.sync_copy(stage, bufB.at[my_rows])
            plsc.subcore_barrier()
            return 0

        lax.fori_loop(0, K // 2, round_pair, 0)

        pltpu.sync_copy(bufA.at[my_rows], stage)
        pltpu.sync_copy(stage, out_hbm.at[cid, my_rows])

    return prop


_prop = _make_prop_kernel()


def kernel(x, edge_index, norm, W1, b1, W2, b2):
    h0 = _mlp(x, W1.T, b1, W2.T, b2)

    h0s = jnp.stack([h0[:, :CH], h0[:, CH:]], axis=0)
    h0s = jnp.pad(h0s, ((0, 0), (0, NPAD - N), (0, 0)))
    srcr = jnp.pad(edge_index[0], (0, EPAD - E)).reshape(NUM_TILES, NCH, CHUNK)
    dstr = jnp.pad(edge_index[1], (0, EPAD - E)).reshape(NUM_TILES, NCH, CHUNK)
    normr = jnp.pad(norm, (0, EPAD - E)).reshape(NUM_TILES, NCH, CHUNK)

    hks = _prop(h0s, srcr, dstr, normr)
    hk = jnp.concatenate([hks[0, :N], hks[1, :N]], axis=1)
    return _log_softmax(hk)


# speedup vs baseline: 13.7732x; 1.0282x over previous
import functools

import jax
import jax.numpy as jnp
from jax import lax
from jax.experimental import pallas as pl
from jax.experimental.pallas import tpu as pltpu
from jax.experimental.pallas import tpu_sc as plsc

N, E, F, H, C, K = 10000, 160000, 256, 512, 64, 8

NUM_TILES = 16
CHUNK = 128
NCH = 81
EPT = NCH * CHUNK
EPAD = NUM_TILES * EPT
NPAD = 10240
ROWS_PER_TILE = NPAD // NUM_TILES
CH = C // 2
VEC = 16


def _mlp(x, W1t, b1, W2t, b2):
    Bm = 2000

    def body(x_ref, w1_ref, b1_ref, w2_ref, b2_ref, o_ref):
        h = jnp.dot(x_ref[...], w1_ref[...],
                    preferred_element_type=jnp.float32,
                    precision=lax.Precision.HIGHEST)
        h = jnp.maximum(h + b1_ref[...], 0.0)
        o = jnp.dot(h, w2_ref[...],
                    preferred_element_type=jnp.float32,
                    precision=lax.Precision.HIGHEST) + b2_ref[...]
        o_ref[0] = o[:, :CH]
        o_ref[1] = o[:, CH:]

    return pl.pallas_call(
        body,
        grid=(N // Bm,),
        in_specs=[
            pl.BlockSpec((Bm, F), lambda i: (i, 0)),
            pl.BlockSpec((F, H), lambda i: (0, 0)),
            pl.BlockSpec((1, H), lambda i: (0, 0)),
            pl.BlockSpec((H, C), lambda i: (0, 0)),
            pl.BlockSpec((1, C), lambda i: (0, 0)),
        ],
        out_specs=pl.BlockSpec((2, Bm, CH), lambda i: (0, i, 0)),
        out_shape=jax.ShapeDtypeStruct((2, NPAD, CH), jnp.float32),
    )(x, W1t, b1.reshape(1, H), W2t, b2.reshape(1, C))


def _log_softmax(h):
    Bm = 2000

    def body(h_ref, o_ref):
        v = jnp.concatenate([h_ref[0], h_ref[1]], axis=1)
        m = jnp.max(v, axis=1, keepdims=True)
        e = jnp.exp(v - m)
        s = jnp.log(jnp.sum(e, axis=1, keepdims=True))
        o_ref[...] = v - m - s

    return pl.pallas_call(
        body,
        grid=(N // Bm,),
        in_specs=[pl.BlockSpec((2, Bm, CH), lambda i: (0, i, 0))],
        out_specs=pl.BlockSpec((Bm, C), lambda i: (i, 0)),
        out_shape=jax.ShapeDtypeStruct((N, C), jnp.float32),
    )(h)


def _make_prop_kernel():
    mesh = plsc.VectorSubcoreMesh(core_axis_name="c", subcore_axis_name="s")

    @functools.partial(
        pl.kernel,
        mesh=mesh,
        compiler_params=pltpu.CompilerParams(use_tc_tiling_on_sc=False),
        out_type=jax.ShapeDtypeStruct((2, NPAD, CH), jnp.float32),
        scratch_types=[
            pltpu.VMEM_SHARED((NPAD, CH), jnp.float32),
            pltpu.VMEM_SHARED((NPAD, CH), jnp.float32),
            pltpu.VMEM((NCH, CHUNK), jnp.int32),
            pltpu.VMEM((NCH, CHUNK), jnp.int32),
            pltpu.VMEM((NCH, CHUNK), jnp.float32),
            pltpu.VMEM((CHUNK, CH), jnp.float32),
            pltpu.VMEM((CHUNK, CH), jnp.float32),
            pltpu.VMEM((CHUNK, CH), jnp.float32),
            pltpu.VMEM((ROWS_PER_TILE, CH), jnp.float32),
            pltpu.SemaphoreType.DMA,
            pltpu.SemaphoreType.DMA,
            pltpu.SemaphoreType.DMA,
            pltpu.SemaphoreType.DMA,
            pltpu.SemaphoreType.DMA,
            pltpu.SemaphoreType.DMA,
        ],
    )
    def prop(h0_hbm, src_hbm, dst_hbm, norm_hbm, out_hbm,
             bufA, bufB, src_v, dst_v, norm_v, rows0, rows1, rows2, stage,
             gsem0, gsem1, gsem2, ssem0, ssem1, ssem2):
        rows_bufs = (rows0, rows1, rows2)
        gsems = (gsem0, gsem1, gsem2)
        ssems = (ssem0, ssem1, ssem2)
        cid = lax.axis_index("c")
        sid = lax.axis_index("s")
        row0 = sid * ROWS_PER_TILE
        my_rows = pl.ds(row0, ROWS_PER_TILE)

        pltpu.sync_copy(src_hbm.at[sid], src_v)
        pltpu.sync_copy(dst_hbm.at[sid], dst_v)
        pltpu.sync_copy(norm_hbm.at[sid], norm_v)

        pltpu.sync_copy(h0_hbm.at[cid, my_rows], stage)
        pltpu.sync_copy(stage, bufA.at[my_rows])

        zvec = jnp.zeros((VEC,), jnp.float32)

        def zrow(i, _):
            for j in range(CH // VEC):
                stage[i, pl.ds(j * VEC, VEC)] = zvec
            return 0

        lax.fori_loop(0, ROWS_PER_TILE, zrow, 0)
        pltpu.sync_copy(stage, bufB.at[my_rows])
        plsc.subcore_barrier()

        vecs_per_row = CH // VEC

        def scale(rows, ci):
            for g in range(CHUNK // VEC):
                nvec = norm_v[ci, pl.ds(g * VEC, VEC)]
                for j in range(VEC * vecs_per_row):
                    r = g * VEC + j // vecs_per_row
                    c0 = (j % vecs_per_row) * VEC
                    nv = nvec[j // vecs_per_row]
                    sl = pl.ds(c0, VEC)
                    rows[r, sl] = rows[r, sl] * nv

        def one_round(rd, wr):
            pltpu.make_async_copy(rd.at[src_v.at[0]], rows0, gsem0).start()

            @pl.loop(0, NCH, step=3)
            def chunk_trio(c):
                for k in range(3):
                    cc = c + k
                    b = k
                    nb = (k + 1) % 3
                    @pl.when(cc >= 2)
                    def _():
                        pltpu.make_async_copy(
                            rows_bufs[nb], wr.at[dst_v.at[cc - 2]],
                            ssems[nb]).wait()
                    @pl.when(cc + 1 < NCH)
                    def _():
                        pltpu.make_async_copy(
                            rd.at[src_v.at[cc + 1]], rows_bufs[nb],
                            gsems[nb]).start()
                    pltpu.make_async_copy(
                        rd.at[src_v.at[cc]], rows_bufs[b], gsems[b]).wait()
                    scale(rows_bufs[b], cc)
                    pltpu.make_async_copy(
                        rows_bufs[b], wr.at[dst_v.at[cc]],
                        ssems[b]).start(add=True)

            pltpu.make_async_copy(
                rows_bufs[(NCH - 2) % 3], wr.at[dst_v.at[NCH - 2]],
                ssems[(NCH - 2) % 3]).wait()
            pltpu.make_async_copy(
                rows_bufs[(NCH - 1) % 3], wr.at[dst_v.at[NCH - 1]],
                ssems[(NCH - 1) % 3]).wait()
            plsc.subcore_barrier()

        def round_pair(rp, _):
            one_round(bufA, bufB)
            pltpu.sync_copy(stage, bufA.at[my_rows])
            plsc.subcore_barrier()
            one_round(bufB, bufA)
            pltpu.sync_copy(stage, bufB.at[my_rows])
            plsc.subcore_barrier()
            return 0

        lax.fori_loop(0, K // 2, round_pair, 0)

        pltpu.sync_copy(bufA.at[my_rows], stage)
        pltpu.sync_copy(stage, out_hbm.at[cid, my_rows])

    return prop


_prop = _make_prop_kernel()


def kernel(x, edge_index, norm, W1, b1, W2, b2):
    h0s = _mlp(x, W1.T, b1, W2.T, b2)

    srcr = jnp.pad(edge_index[0], (0, EPAD - E)).reshape(NUM_TILES, NCH, CHUNK)
    dstr = jnp.pad(edge_index[1], (0, EPAD - E)).reshape(NUM_TILES, NCH, CHUNK)
    normr = jnp.pad(norm, (0, EPAD - E)).reshape(NUM_TILES, NCH, CHUNK)

    hks = _prop(h0s, srcr, dstr, normr)
    return _log_softmax(hks)


# speedup vs baseline: 14.7135x; 1.0683x over previous
import functools

import jax
import jax.numpy as jnp
from jax import lax
from jax.experimental import pallas as pl
from jax.experimental.pallas import tpu as pltpu
from jax.experimental.pallas import tpu_sc as plsc

N, E, F, H, C, K = 10000, 160000, 256, 512, 64, 8

NUM_TILES = 16
CHUNK = 128
NCH = 80
EPT = NCH * CHUNK
EPAD = NUM_TILES * EPT
NPAD = 10240
ROWS_PER_TILE = NPAD // NUM_TILES
CH = C // 2
VEC = 16


def _mlp(x, W1t, b1, W2t, b2):
    Bm = 2000

    def body(x_ref, w1_ref, b1_ref, w2_ref, b2_ref, o_ref):
        h = jnp.dot(x_ref[...], w1_ref[...],
                    preferred_element_type=jnp.float32,
                    precision=lax.Precision.HIGHEST)
        h = jnp.maximum(h + b1_ref[...], 0.0)
        o = jnp.dot(h, w2_ref[...],
                    preferred_element_type=jnp.float32,
                    precision=lax.Precision.HIGHEST) + b2_ref[...]
        o_ref[0] = o[:, :CH]
        o_ref[1] = o[:, CH:]

    return pl.pallas_call(
        body,
        grid=(N // Bm,),
        in_specs=[
            pl.BlockSpec((Bm, F), lambda i: (i, 0)),
            pl.BlockSpec((F, H), lambda i: (0, 0)),
            pl.BlockSpec((1, H), lambda i: (0, 0)),
            pl.BlockSpec((H, C), lambda i: (0, 0)),
            pl.BlockSpec((1, C), lambda i: (0, 0)),
        ],
        out_specs=pl.BlockSpec((2, Bm, CH), lambda i: (0, i, 0)),
        out_shape=jax.ShapeDtypeStruct((2, NPAD, CH), jnp.float32),
    )(x, W1t, b1.reshape(1, H), W2t, b2.reshape(1, C))


def _log_softmax(h):
    Bm = 2000

    def body(h_ref, o_ref):
        v = jnp.concatenate([h_ref[0], h_ref[1]], axis=1)
        m = jnp.max(v, axis=1, keepdims=True)
        e = jnp.exp(v - m)
        s = jnp.log(jnp.sum(e, axis=1, keepdims=True))
        o_ref[...] = v - m - s

    return pl.pallas_call(
        body,
        grid=(N // Bm,),
        in_specs=[pl.BlockSpec((2, Bm, CH), lambda i: (0, i, 0))],
        out_specs=pl.BlockSpec((Bm, C), lambda i: (i, 0)),
        out_shape=jax.ShapeDtypeStruct((N, C), jnp.float32),
    )(h)


def _make_prop_kernel():
    mesh = plsc.VectorSubcoreMesh(core_axis_name="c", subcore_axis_name="s")

    @functools.partial(
        pl.kernel,
        mesh=mesh,
        compiler_params=pltpu.CompilerParams(use_tc_tiling_on_sc=False),
        out_type=jax.ShapeDtypeStruct((2, NPAD, CH), jnp.float32),
        scratch_types=[
            pltpu.VMEM_SHARED((NPAD, CH), jnp.float32),
            pltpu.VMEM_SHARED((NPAD, CH), jnp.float32),
            pltpu.VMEM((NCH, CHUNK), jnp.int32),
            pltpu.VMEM((NCH, CHUNK), jnp.int32),
            pltpu.VMEM((NCH, CHUNK), jnp.float32),
            pltpu.VMEM((CHUNK, CH), jnp.float32),
            pltpu.VMEM((CHUNK, CH), jnp.float32),
            pltpu.VMEM((CHUNK, CH), jnp.float32),
            pltpu.VMEM((CHUNK, CH), jnp.float32),
            pltpu.VMEM((ROWS_PER_TILE, CH), jnp.float32),
            pltpu.SemaphoreType.DMA,
            pltpu.SemaphoreType.DMA,
            pltpu.SemaphoreType.DMA,
            pltpu.SemaphoreType.DMA,
            pltpu.SemaphoreType.DMA,
            pltpu.SemaphoreType.DMA,
            pltpu.SemaphoreType.DMA,
            pltpu.SemaphoreType.DMA,
        ],
    )
    def prop(h0_hbm, src_hbm, dst_hbm, norm_hbm, out_hbm,
             bufA, bufB, src_v, dst_v, norm_v, rows0, rows1, rows2, rows3,
             stage, gsem0, gsem1, gsem2, gsem3, ssem0, ssem1, ssem2, ssem3):
        rows_bufs = (rows0, rows1, rows2, rows3)
        gsems = (gsem0, gsem1, gsem2, gsem3)
        ssems = (ssem0, ssem1, ssem2, ssem3)
        cid = lax.axis_index("c")
        sid = lax.axis_index("s")
        row0 = sid * ROWS_PER_TILE
        my_rows = pl.ds(row0, ROWS_PER_TILE)

        pltpu.sync_copy(src_hbm.at[sid], src_v)
        pltpu.sync_copy(dst_hbm.at[sid], dst_v)
        pltpu.sync_copy(norm_hbm.at[sid], norm_v)

        pltpu.sync_copy(h0_hbm.at[cid, my_rows], stage)
        pltpu.sync_copy(stage, bufA.at[my_rows])

        zvec = jnp.zeros((VEC,), jnp.float32)

        def zrow(i, _):
            for j in range(CH // VEC):
                stage[i, pl.ds(j * VEC, VEC)] = zvec
            return 0

        lax.fori_loop(0, ROWS_PER_TILE, zrow, 0)
        pltpu.sync_copy(stage, bufB.at[my_rows])
        plsc.subcore_barrier()

        vecs_per_row = CH // VEC

        def scale(rows, ci):
            for g in range(CHUNK // VEC):
                nvec = norm_v[ci, pl.ds(g * VEC, VEC)]
                for j in range(VEC * vecs_per_row):
                    r = g * VEC + j // vecs_per_row
                    c0 = (j % vecs_per_row) * VEC
                    nv = nvec[j // vecs_per_row]
                    sl = pl.ds(c0, VEC)
                    rows[r, sl] = rows[r, sl] * nv

        def one_round(rd, wr):
            pltpu.make_async_copy(rd.at[src_v.at[0]], rows0, gsem0).start()
            pltpu.make_async_copy(rd.at[src_v.at[1]], rows1, gsem1).start()

            @pl.loop(0, NCH, step=4)
            def chunk_quad(c):
                for k in range(4):
                    cc = c + k
                    b = k
                    pf = (k + 2) % 4
                    @pl.when(cc >= 2)
                    def _():
                        pltpu.make_async_copy(
                            rows_bufs[pf], wr.at[dst_v.at[cc - 2]],
                            ssems[pf]).wait()
                    @pl.when(cc + 2 < NCH)
                    def _():
                        pltpu.make_async_copy(
                            rd.at[src_v.at[cc + 2]], rows_bufs[pf],
                            gsems[pf]).start()
                    pltpu.make_async_copy(
                        rd.at[src_v.at[cc]], rows_bufs[b], gsems[b]).wait()
                    scale(rows_bufs[b], cc)
                    pltpu.make_async_copy(
                        rows_bufs[b], wr.at[dst_v.at[cc]],
                        ssems[b]).start(add=True)

            pltpu.make_async_copy(
                rows_bufs[(NCH - 2) % 4], wr.at[dst_v.at[NCH - 2]],
                ssems[(NCH - 2) % 4]).wait()
            pltpu.make_async_copy(
                rows_bufs[(NCH - 1) % 4], wr.at[dst_v.at[NCH - 1]],
                ssems[(NCH - 1) % 4]).wait()
            plsc.subcore_barrier()

        def round_pair(rp, _):
            one_round(bufA, bufB)
            pltpu.sync_copy(stage, bufA.at[my_rows])
            plsc.subcore_barrier()
            one_round(bufB, bufA)
            pltpu.sync_copy(stage, bufB.at[my_rows])
            plsc.subcore_barrier()
            return 0

        lax.fori_loop(0, K // 2, round_pair, 0)

        pltpu.sync_copy(bufA.at[my_rows], stage)
        pltpu.sync_copy(stage, out_hbm.at[cid, my_rows])

    return prop


_prop = _make_prop_kernel()


def kernel(x, edge_index, norm, W1, b1, W2, b2):
    h0s = _mlp(x, W1.T, b1, W2.T, b2)

    srcr = jnp.pad(edge_index[0], (0, EPAD - E)).reshape(NUM_TILES, NCH, CHUNK)
    dstr = jnp.pad(edge_index[1], (0, EPAD - E)).reshape(NUM_TILES, NCH, CHUNK)
    normr = jnp.pad(norm, (0, EPAD - E)).reshape(NUM_TILES, NCH, CHUNK)

    hks = _prop(h0s, srcr, dstr, normr)
    return _log_softmax(hks)


# speedup vs baseline: 16.2568x; 1.1049x over previous
import functools

import jax
import jax.numpy as jnp
from jax import lax
from jax.experimental import pallas as pl
from jax.experimental.pallas import tpu as pltpu
from jax.experimental.pallas import tpu_sc as plsc

N, E, F, H, C, K = 10000, 160000, 256, 512, 64, 8

NUM_TILES = 16
CHUNK = 128
NCH = 80
EPT = NCH * CHUNK
EPAD = NUM_TILES * EPT
NPAD = 10240
ROWS_PER_TILE = NPAD // NUM_TILES
CH = C // 2
VEC = 16


def _mlp(x, W1t, b1, W2t, b2):
    Bm = 2000

    def body(x_ref, w1_ref, b1_ref, w2_ref, b2_ref, o_ref):
        h = jnp.dot(x_ref[...], w1_ref[...],
                    preferred_element_type=jnp.float32,
                    precision=lax.Precision.DEFAULT)
        h = jnp.maximum(h + b1_ref[...], 0.0)
        o = jnp.dot(h, w2_ref[...],
                    preferred_element_type=jnp.float32,
                    precision=lax.Precision.DEFAULT) + b2_ref[...]
        o_ref[0] = o[:, :CH]
        o_ref[1] = o[:, CH:]

    return pl.pallas_call(
        body,
        grid=(N // Bm,),
        in_specs=[
            pl.BlockSpec((Bm, F), lambda i: (i, 0)),
            pl.BlockSpec((F, H), lambda i: (0, 0)),
            pl.BlockSpec((1, H), lambda i: (0, 0)),
            pl.BlockSpec((H, C), lambda i: (0, 0)),
            pl.BlockSpec((1, C), lambda i: (0, 0)),
        ],
        out_specs=pl.BlockSpec((2, Bm, CH), lambda i: (0, i, 0)),
        out_shape=jax.ShapeDtypeStruct((2, NPAD, CH), jnp.float32),
    )(x, W1t, b1.reshape(1, H), W2t, b2.reshape(1, C))


def _log_softmax(h):
    Bm = 2000

    def body(h_ref, o_ref):
        v = jnp.concatenate([h_ref[0], h_ref[1]], axis=1)
        m = jnp.max(v, axis=1, keepdims=True)
        e = jnp.exp(v - m)
        s = jnp.log(jnp.sum(e, axis=1, keepdims=True))
        o_ref[...] = v - m - s

    return pl.pallas_call(
        body,
        grid=(N // Bm,),
        in_specs=[pl.BlockSpec((2, Bm, CH), lambda i: (0, i, 0))],
        out_specs=pl.BlockSpec((Bm, C), lambda i: (i, 0)),
        out_shape=jax.ShapeDtypeStruct((N, C), jnp.float32),
    )(h)


def _make_prop_kernel():
    mesh = plsc.VectorSubcoreMesh(core_axis_name="c", subcore_axis_name="s")

    @functools.partial(
        pl.kernel,
        mesh=mesh,
        compiler_params=pltpu.CompilerParams(use_tc_tiling_on_sc=False),
        out_type=jax.ShapeDtypeStruct((2, NPAD, CH), jnp.float32),
        scratch_types=[
            pltpu.VMEM_SHARED((NPAD, CH), jnp.float32),
            pltpu.VMEM_SHARED((NPAD, CH), jnp.float32),
            pltpu.VMEM((NCH, CHUNK), jnp.int32),
            pltpu.VMEM((NCH, CHUNK), jnp.int32),
            pltpu.VMEM((NCH, CHUNK), jnp.float32),
            pltpu.VMEM((CHUNK, CH), jnp.float32),
            pltpu.VMEM((CHUNK, CH), jnp.float32),
            pltpu.VMEM((CHUNK, CH), jnp.float32),
            pltpu.VMEM((CHUNK, CH), jnp.float32),
            pltpu.VMEM((ROWS_PER_TILE, CH), jnp.float32),
            pltpu.SemaphoreType.DMA,
            pltpu.SemaphoreType.DMA,
            pltpu.SemaphoreType.DMA,
            pltpu.SemaphoreType.DMA,
            pltpu.SemaphoreType.DMA,
            pltpu.SemaphoreType.DMA,
            pltpu.SemaphoreType.DMA,
            pltpu.SemaphoreType.DMA,
        ],
    )
    def prop(h0_hbm, src_hbm, dst_hbm, norm_hbm, out_hbm,
             bufA, bufB, src_v, dst_v, norm_v, rows0, rows1, rows2, rows3,
             stage, gsem0, gsem1, gsem2, gsem3, ssem0, ssem1, ssem2, ssem3):
        rows_bufs = (rows0, rows1, rows2, rows3)
        gsems = (gsem0, gsem1, gsem2, gsem3)
        ssems = (ssem0, ssem1, ssem2, ssem3)
        cid = lax.axis_index("c")
        sid = lax.axis_index("s")
        row0 = sid * ROWS_PER_TILE
        my_rows = pl.ds(row0, ROWS_PER_TILE)

        pltpu.sync_copy(src_hbm.at[sid], src_v)
        pltpu.sync_copy(dst_hbm.at[sid], dst_v)
        pltpu.sync_copy(norm_hbm.at[sid], norm_v)

        pltpu.sync_copy(h0_hbm.at[cid, my_rows], stage)
        pltpu.sync_copy(stage, bufA.at[my_rows])

        zvec = jnp.zeros((VEC,), jnp.float32)

        def zrow(i, _):
            for j in range(CH // VEC):
                stage[i, pl.ds(j * VEC, VEC)] = zvec
            return 0

        lax.fori_loop(0, ROWS_PER_TILE, zrow, 0)
        pltpu.sync_copy(stage, bufB.at[my_rows])
        plsc.subcore_barrier()

        vecs_per_row = CH // VEC

        def scale(rows, ci):
            for g in range(CHUNK // VEC):
                nvec = norm_v[ci, pl.ds(g * VEC, VEC)]
                for j in range(VEC * vecs_per_row):
                    r = g * VEC + j // vecs_per_row
                    c0 = (j % vecs_per_row) * VEC
                    nv = nvec[j // vecs_per_row]
                    sl = pl.ds(c0, VEC)
                    rows[r, sl] = rows[r, sl] * nv

        def one_round(rd, wr):
            pltpu.make_async_copy(rd.at[src_v.at[0]], rows0, gsem0).start()
            pltpu.make_async_copy(rd.at[src_v.at[1]], rows1, gsem1).start()

            @pl.loop(0, NCH, step=4)
            def chunk_quad(c):
                for k in range(4):
                    cc = c + k
                    b = k
                    pf = (k + 2) % 4
                    @pl.when(cc >= 2)
                    def _():
                        pltpu.make_async_copy(
                            rows_bufs[pf], wr.at[dst_v.at[cc - 2]],
                            ssems[pf]).wait()
                    @pl.when(cc + 2 < NCH)
                    def _():
                        pltpu.make_async_copy(
                            rd.at[src_v.at[cc + 2]], rows_bufs[pf],
                            gsems[pf]).start()
                    pltpu.make_async_copy(
                        rd.at[src_v.at[cc]], rows_bufs[b], gsems[b]).wait()
                    scale(rows_bufs[b], cc)
                    pltpu.make_async_copy(
                        rows_bufs[b], wr.at[dst_v.at[cc]],
                        ssems[b]).start(add=True)

            pltpu.make_async_copy(
                rows_bufs[(NCH - 2) % 4], wr.at[dst_v.at[NCH - 2]],
                ssems[(NCH - 2) % 4]).wait()
            pltpu.make_async_copy(
                rows_bufs[(NCH - 1) % 4], wr.at[dst_v.at[NCH - 1]],
                ssems[(NCH - 1) % 4]).wait()
            plsc.subcore_barrier()

        def round_pair(rp, _):
            one_round(bufA, bufB)
            pltpu.sync_copy(stage, bufA.at[my_rows])
            plsc.subcore_barrier()
            one_round(bufB, bufA)
            pltpu.sync_copy(stage, bufB.at[my_rows])
            plsc.subcore_barrier()
            return 0

        lax.fori_loop(0, K // 2, round_pair, 0)

        pltpu.sync_copy(bufA.at[my_rows], stage)
        pltpu.sync_copy(stage, out_hbm.at[cid, my_rows])

    return prop


_prop = _make_prop_kernel()


def kernel(x, edge_index, norm, W1, b1, W2, b2):
    h0s = _mlp(x, W1.T, b1, W2.T, b2)

    srcr = jnp.pad(edge_index[0], (0, EPAD - E)).reshape(NUM_TILES, NCH, CHUNK)
    dstr = jnp.pad(edge_index[1], (0, EPAD - E)).reshape(NUM_TILES, NCH, CHUNK)
    normr = jnp.pad(norm, (0, EPAD - E)).reshape(NUM_TILES, NCH, CHUNK)

    hks = _prop(h0s, srcr, dstr, normr)
    return _log_softmax(hks)
